# baseline (device time: 38532 ns/iter reference)
import jax
import jax.numpy as jnp
from jax import lax
from jax.experimental import pallas as pl
from jax.experimental.pallas import tpu as pltpu

N_DEV = 4
B = 2
SQ = 128
SKV_SH = 128
H_LOC = 4
DH = 64
D_MODEL = 512
QBLK = 64


def kernel(x, Wq, K_ext, V_ext, Wo):
    xb = x.astype(jnp.bfloat16)
    wqb = Wq.astype(jnp.bfloat16)
    wob = Wo.astype(jnp.bfloat16)
    kt = jnp.transpose(K_ext, (2, 0, 1, 3)).astype(jnp.bfloat16)
    kt = kt.reshape(N_DEV * H_LOC, B * SKV_SH, DH)
    vt = jnp.transpose(V_ext, (2, 0, 1, 3)).astype(jnp.bfloat16)
    vt = vt.reshape(N_DEV * H_LOC, B * SKV_SH, DH)

    def body(x_ref, wq_ref, kt_ref, vt_ref, wo_ref, out_ref,
             k_buf, v_buf, p_ref, acc_buf,
             ksend, krecv, vsend, vrecv, psend, precv, copy_sems):
        my_i = lax.axis_index("i")

        barrier = pltpu.get_barrier_semaphore()
        for off in range(1, N_DEV):
            peer = lax.rem(my_i + off, N_DEV)
            pl.semaphore_signal(barrier, inc=1, device_id=(peer,),
                                device_id_type=pl.DeviceIdType.MESH)
        pl.semaphore_wait(barrier, N_DEV - 1)

        sends = []
        for off in range(1, N_DEV):
            dst = lax.rem(my_i + off, N_DEV)
            for buf, src_ref, ssem, rsem in (
                (k_buf, kt_ref, ksend, krecv),
                (v_buf, vt_ref, vsend, vrecv),
            ):
                rdma = pltpu.make_async_remote_copy(
                    src_ref=src_ref.at[pl.ds(dst * H_LOC, H_LOC)],
                    dst_ref=buf.at[my_i],
                    send_sem=ssem.at[off - 1],
                    recv_sem=rsem.at[my_i],
                    device_id=(dst,),
                    device_id_type=pl.DeviceIdType.MESH,
                )
                rdma.start()
                sends.append(rdma)
        kcopy = pltpu.make_async_copy(
            kt_ref.at[pl.ds(my_i * H_LOC, H_LOC)], k_buf.at[my_i],
            copy_sems.at[0])
        vcopy = pltpu.make_async_copy(
            vt_ref.at[pl.ds(my_i * H_LOC, H_LOC)], v_buf.at[my_i],
            copy_sems.at[1])
        kcopy.start()
        vcopy.start()

        q = [jnp.dot(x_ref[b], wq_ref[...],
                     preferred_element_type=jnp.float32).astype(jnp.bfloat16)
             for b in range(B)]

        kcopy.wait()
        vcopy.wait()
        for off in range(1, N_DEV):
            src = lax.rem(my_i + off, N_DEV)
            for buf, src_ref, ssem, rsem in (
                (k_buf, kt_ref, ksend, krecv),
                (v_buf, vt_ref, vsend, vrecv),
            ):
                pltpu.make_async_remote_copy(
                    src_ref=src_ref.at[pl.ds(0, H_LOC)],
                    dst_ref=buf.at[src],
                    send_sem=ssem.at[off - 1],
                    recv_sem=rsem.at[src],
                    device_id=(src,),
                    device_id_type=pl.DeviceIdType.MESH,
                ).wait_recv()

        qb = lax.broadcasted_iota(jnp.int32, (SQ, N_DEV * SKV_SH), 0) // QBLK
        kb = lax.broadcasted_iota(jnp.int32, (SQ, N_DEV * SKV_SH), 1) // QBLK
        mask = (qb == kb) | (kb == 0) | (((qb + kb) % 3) == 0)
        for b in range(B):
            ctx_cols = []
            for h in range(H_LOC):
                qh = q[b][:, h * DH:(h + 1) * DH]
                kh = jnp.concatenate(
                    [k_buf[s, h, b * SKV_SH:(b + 1) * SKV_SH, :]
                     for s in range(N_DEV)], axis=0)
                vh = jnp.concatenate(
                    [v_buf[s, h, b * SKV_SH:(b + 1) * SKV_SH, :]
                     for s in range(N_DEV)], axis=0)
                s_full = lax.dot_general(
                    qh, kh, (((1,), (1,)), ((), ())),
                    preferred_element_type=jnp.float32) * 0.125
                s_full = jnp.where(mask, s_full, -1e9)
                m = jnp.max(s_full, axis=1, keepdims=True)
                w = jnp.exp(s_full - m)
                w = w / jnp.sum(w, axis=1, keepdims=True)
                ctx_cols.append(
                    jnp.dot(w.astype(jnp.bfloat16), vh,
                            preferred_element_type=jnp.float32)
                    .astype(jnp.bfloat16))
            ctx = jnp.concatenate(ctx_cols, axis=1)
            p_ref[b] = jnp.dot(ctx, wo_ref[...],
                               preferred_element_type=jnp.float32)

        for off in range(1, N_DEV):
            dst = lax.rem(my_i + off, N_DEV)
            rdma = pltpu.make_async_remote_copy(
                src_ref=p_ref, dst_ref=acc_buf.at[my_i],
                send_sem=psend.at[off - 1], recv_sem=precv.at[my_i],
                device_id=(dst,), device_id_type=pl.DeviceIdType.MESH)
            rdma.start()
            sends.append(rdma)
        pcopy = pltpu.make_async_copy(p_ref, acc_buf.at[my_i], copy_sems.at[2])
        pcopy.start()
        pcopy.wait()
        for off in range(1, N_DEV):
            src = lax.rem(my_i + off, N_DEV)
            pltpu.make_async_remote_copy(
                src_ref=p_ref, dst_ref=acc_buf.at[src],
                send_sem=psend.at[off - 1], recv_sem=precv.at[src],
                device_id=(src,), device_id_type=pl.DeviceIdType.MESH,
            ).wait_recv()

        out_ref[...] = acc_buf[0] + acc_buf[1] + acc_buf[2] + acc_buf[3]

        for rdma in sends:
            rdma.wait_send()

    return pl.pallas_call(
        body,
        out_shape=jax.ShapeDtypeStruct((B, SQ, D_MODEL), jnp.float32),
        in_specs=[pl.BlockSpec(memory_space=pltpu.VMEM)] * 5,
        out_specs=pl.BlockSpec(memory_space=pltpu.VMEM),
        scratch_shapes=[
            pltpu.VMEM((N_DEV, H_LOC, B * SKV_SH, DH), jnp.bfloat16),
            pltpu.VMEM((N_DEV, H_LOC, B * SKV_SH, DH), jnp.bfloat16),
            pltpu.VMEM((B, SQ, D_MODEL), jnp.float32),
            pltpu.VMEM((N_DEV, B, SQ, D_MODEL), jnp.float32),
            pltpu.SemaphoreType.DMA((N_DEV - 1,)),
            pltpu.SemaphoreType.DMA((N_DEV,)),
            pltpu.SemaphoreType.DMA((N_DEV - 1,)),
            pltpu.SemaphoreType.DMA((N_DEV,)),
            pltpu.SemaphoreType.DMA((N_DEV - 1,)),
            pltpu.SemaphoreType.DMA((N_DEV,)),
            pltpu.SemaphoreType.DMA((3,)),
        ],
        compiler_params=pltpu.CompilerParams(collective_id=0),
    )(xb, wqb, kt, vt, wob)


# device time: 32903 ns/iter; 1.1711x vs baseline; 1.1711x over previous
import jax
import jax.numpy as jnp
from jax import lax
from jax.experimental import pallas as pl
from jax.experimental.pallas import tpu as pltpu

N_DEV = 4
B = 2
SQ = 128
SKV_SH = 128
H_LOC = 4
DH = 64
D_MODEL = 512
QBLK = 64


def kernel(x, Wq, K_ext, V_ext, Wo):
    xb = x.astype(jnp.bfloat16)
    wqb = Wq.astype(jnp.bfloat16)
    wob = Wo.astype(jnp.bfloat16)
    kt = jnp.transpose(K_ext, (2, 0, 1, 3)).astype(jnp.bfloat16)
    kt = kt.reshape(N_DEV * H_LOC, B * SKV_SH, DH)
    vt = jnp.transpose(V_ext, (2, 0, 1, 3)).astype(jnp.bfloat16)
    vt = vt.reshape(N_DEV * H_LOC, B * SKV_SH, DH)

    def body(x_ref, wq_ref, kt_ref, vt_ref, wo_ref, out_ref,
             k_buf, v_buf, p_ref, acc_buf,
             ksend, krecv, vsend, vrecv, psend, precv, copy_sems):
        my_i = lax.axis_index("i")

        barrier = pltpu.get_barrier_semaphore()
        for off in range(1, N_DEV):
            peer = lax.rem(my_i + off, N_DEV)
            pl.semaphore_signal(barrier, inc=1, device_id=(peer,),
                                device_id_type=pl.DeviceIdType.MESH)
        pl.semaphore_wait(barrier, N_DEV - 1)

        sends = []
        for off in range(1, N_DEV):
            dst = lax.rem(my_i + off, N_DEV)
            for buf, src_ref, ssem, rsem in (
                (k_buf, kt_ref, ksend, krecv),
                (v_buf, vt_ref, vsend, vrecv),
            ):
                rdma = pltpu.make_async_remote_copy(
                    src_ref=src_ref.at[pl.ds(dst * H_LOC, H_LOC)],
                    dst_ref=buf.at[my_i],
                    send_sem=ssem.at[off - 1],
                    recv_sem=rsem.at[my_i],
                    device_id=(dst,),
                    device_id_type=pl.DeviceIdType.MESH,
                )
                rdma.start()
                sends.append(rdma)
        kcopy = pltpu.make_async_copy(
            kt_ref.at[pl.ds(my_i * H_LOC, H_LOC)], k_buf.at[my_i],
            copy_sems.at[0])
        vcopy = pltpu.make_async_copy(
            vt_ref.at[pl.ds(my_i * H_LOC, H_LOC)], v_buf.at[my_i],
            copy_sems.at[1])
        kcopy.start()
        vcopy.start()

        q = [jnp.dot(x_ref[b], wq_ref[...],
                     preferred_element_type=jnp.float32).astype(jnp.bfloat16)
             for b in range(B)]

        kcopy.wait()
        vcopy.wait()
        for off in range(1, N_DEV):
            src = lax.rem(my_i + off, N_DEV)
            for buf, src_ref, ssem, rsem in (
                (k_buf, kt_ref, ksend, krecv),
                (v_buf, vt_ref, vsend, vrecv),
            ):
                pltpu.make_async_remote_copy(
                    src_ref=src_ref.at[pl.ds(0, H_LOC)],
                    dst_ref=buf.at[src],
                    send_sem=ssem.at[off - 1],
                    recv_sem=rsem.at[src],
                    device_id=(src,),
                    device_id_type=pl.DeviceIdType.MESH,
                ).wait_recv()

        qb = lax.broadcasted_iota(jnp.int32, (SQ, N_DEV * SKV_SH), 0) // QBLK
        kb = lax.broadcasted_iota(jnp.int32, (SQ, N_DEV * SKV_SH), 1) // QBLK
        mask = (qb == kb) | (kb == 0) | (((qb + kb) % 3) == 0)
        for b in range(B):
            ctx_cols = []
            for h in range(H_LOC):
                qh = q[b][:, h * DH:(h + 1) * DH]
                kh = jnp.concatenate(
                    [k_buf[s, h, b * SKV_SH:(b + 1) * SKV_SH, :]
                     for s in range(N_DEV)], axis=0)
                vh = jnp.concatenate(
                    [v_buf[s, h, b * SKV_SH:(b + 1) * SKV_SH, :]
                     for s in range(N_DEV)], axis=0)
                s_full = lax.dot_general(
                    qh, kh, (((1,), (1,)), ((), ())),
                    preferred_element_type=jnp.float32) * 0.125
                s_full = jnp.where(mask, s_full, -1e9)
                m = jnp.max(s_full, axis=1, keepdims=True)
                w = jnp.exp(s_full - m)
                w = w / jnp.sum(w, axis=1, keepdims=True)
                ctx_cols.append(
                    jnp.dot(w.astype(jnp.bfloat16), vh,
                            preferred_element_type=jnp.float32)
                    .astype(jnp.bfloat16))
            ctx = jnp.concatenate(ctx_cols, axis=1)
            p_ref[b] = jnp.dot(ctx, wo_ref[...],
                               preferred_element_type=jnp.float32
                               ).astype(jnp.bfloat16)

        for off in range(1, N_DEV):
            dst = lax.rem(my_i + off, N_DEV)
            rdma = pltpu.make_async_remote_copy(
                src_ref=p_ref, dst_ref=acc_buf.at[my_i],
                send_sem=psend.at[off - 1], recv_sem=precv.at[my_i],
                device_id=(dst,), device_id_type=pl.DeviceIdType.MESH)
            rdma.start()
            sends.append(rdma)
        pcopy = pltpu.make_async_copy(p_ref, acc_buf.at[my_i], copy_sems.at[2])
        pcopy.start()
        pcopy.wait()
        for off in range(1, N_DEV):
            src = lax.rem(my_i + off, N_DEV)
            pltpu.make_async_remote_copy(
                src_ref=p_ref, dst_ref=acc_buf.at[src],
                send_sem=psend.at[off - 1], recv_sem=precv.at[src],
                device_id=(src,), device_id_type=pl.DeviceIdType.MESH,
            ).wait_recv()

        out_ref[...] = (
            acc_buf[0].astype(jnp.float32) + acc_buf[1].astype(jnp.float32)
            + acc_buf[2].astype(jnp.float32) + acc_buf[3].astype(jnp.float32))

        for rdma in sends:
            rdma.wait_send()

    return pl.pallas_call(
        body,
        out_shape=jax.ShapeDtypeStruct((B, SQ, D_MODEL), jnp.float32),
        in_specs=[pl.BlockSpec(memory_space=pltpu.VMEM)] * 5,
        out_specs=pl.BlockSpec(memory_space=pltpu.VMEM),
        scratch_shapes=[
            pltpu.VMEM((N_DEV, H_LOC, B * SKV_SH, DH), jnp.bfloat16),
            pltpu.VMEM((N_DEV, H_LOC, B * SKV_SH, DH), jnp.bfloat16),
            pltpu.VMEM((B, SQ, D_MODEL), jnp.bfloat16),
            pltpu.VMEM((N_DEV, B, SQ, D_MODEL), jnp.bfloat16),
            pltpu.SemaphoreType.DMA((N_DEV - 1,)),
            pltpu.SemaphoreType.DMA((N_DEV,)),
            pltpu.SemaphoreType.DMA((N_DEV - 1,)),
            pltpu.SemaphoreType.DMA((N_DEV,)),
            pltpu.SemaphoreType.DMA((N_DEV - 1,)),
            pltpu.SemaphoreType.DMA((N_DEV,)),
            pltpu.SemaphoreType.DMA((3,)),
        ],
        compiler_params=pltpu.CompilerParams(collective_id=0),
    )(xb, wqb, kt, vt, wob)
